# Initial kernel scaffold; baseline (speedup 1.0000x reference)
import jax
import jax.numpy as jnp
from jax.experimental import pallas as pl


def kernel(x, embed_weight, fc_W, fc_b):
    raise NotImplementedError("write your pallas kernel here")


# speedup vs baseline: 3.5771x; 3.5771x over previous
import functools

import jax
import jax.numpy as jnp
from jax import lax
from jax.experimental import pallas as pl
from jax.experimental.pallas import tpu as pltpu
from jax.experimental.pallas import tpu_sc as plsc

VOCAB = 100
DIM = 128

NC = 2
NS = 16
NW = NC * NS

B_TOKENS = 4096 * 200
B_PER_W = B_TOKENS // NW
CHUNK = 128
N_CHUNKS = B_PER_W // CHUNK


def _table_body(ew_ref, w_ref, b_ref, out_ref):
    ew = ew_ref[...]
    w = w_ref[...]
    out_ref[...] = (
        lax.dot_general(ew, w, (((1,), (1,)), ((), ())),
                        preferred_element_type=jnp.float32)
        + b_ref[...]
    )


def _fused_table(embed_weight, fc_W, fc_b):
    return pl.pallas_call(
        _table_body,
        out_shape=jax.ShapeDtypeStruct((VOCAB, DIM), jnp.float32),
    )(embed_weight, fc_W, fc_b.reshape(1, DIM))


_sc_mesh = plsc.VectorSubcoreMesh(
    core_axis_name="c", subcore_axis_name="s", num_cores=NC, num_subcores=NS
)


@functools.partial(
    pl.kernel,
    out_type=jax.ShapeDtypeStruct((B_TOKENS, DIM), jnp.float32),
    mesh=_sc_mesh,
    scratch_types=[
        pltpu.VMEM((B_PER_W,), jnp.int32),
        pltpu.VMEM((CHUNK, DIM), jnp.float32),
        pltpu.SemaphoreType.DMA,
    ],
)
def _sc_gather(table_hbm, idx_hbm, out_hbm, idx_v, rows_v, sem):
    wid = lax.axis_index("s") * NC + lax.axis_index("c")
    base = wid * B_PER_W
    pltpu.sync_copy(idx_hbm.at[pl.ds(base, B_PER_W)], idx_v)

    def body(g, carry):
        row0 = pl.multiple_of(g * CHUNK, CHUNK)
        pltpu.async_copy(
            table_hbm.at[idx_v.at[pl.ds(row0, CHUNK)]], rows_v, sem
        ).wait()
        pltpu.sync_copy(rows_v, out_hbm.at[pl.ds(base + row0, CHUNK)])
        return carry

    lax.fori_loop(0, N_CHUNKS, body, 0)


def kernel(x, embed_weight, fc_W, fc_b):
    table = _fused_table(embed_weight, fc_W, fc_b)
    idx = x.reshape(-1).astype(jnp.int32)
    out = _sc_gather(table, idx)
    return out.reshape(x.shape[0], x.shape[1], DIM)


# speedup vs baseline: 3.7122x; 1.0378x over previous
import functools

import jax
import jax.numpy as jnp
from jax import lax
from jax.experimental import pallas as pl
from jax.experimental.pallas import tpu as pltpu
from jax.experimental.pallas import tpu_sc as plsc

VOCAB = 100
DIM = 128

NC = 2
NS = 16
NW = NC * NS

B_TOKENS = 4096 * 200
B_PER_W = B_TOKENS // NW
CHUNK = 128
N_CHUNKS = B_PER_W // CHUNK
NBUF = 4


def _table_body(ew_ref, w_ref, b_ref, out_ref):
    ew = ew_ref[...]
    w = w_ref[...]
    out_ref[...] = (
        lax.dot_general(ew, w, (((1,), (1,)), ((), ())),
                        preferred_element_type=jnp.float32)
        + b_ref[...]
    )


def _fused_table(embed_weight, fc_W, fc_b):
    return pl.pallas_call(
        _table_body,
        out_shape=jax.ShapeDtypeStruct((VOCAB, DIM), jnp.float32),
    )(embed_weight, fc_W, fc_b.reshape(1, DIM))


_sc_mesh = plsc.VectorSubcoreMesh(
    core_axis_name="c", subcore_axis_name="s", num_cores=NC, num_subcores=NS
)


@functools.partial(
    pl.kernel,
    out_type=jax.ShapeDtypeStruct((B_TOKENS, DIM), jnp.float32),
    mesh=_sc_mesh,
    scratch_types=[
        pltpu.VMEM((B_PER_W,), jnp.int32),
        pltpu.VMEM((NBUF, CHUNK, DIM), jnp.float32),
        pltpu.SemaphoreType.DMA((NBUF,)),
        pltpu.SemaphoreType.DMA((NBUF,)),
    ],
)
def _sc_gather(table_hbm, idx_hbm, out_hbm, idx_v, rows_v, sem_in, sem_out):
    wid = lax.axis_index("s") * NC + lax.axis_index("c")
    base = wid * B_PER_W
    pltpu.sync_copy(idx_hbm.at[pl.ds(base, B_PER_W)], idx_v)

    def gather_copy(g, b):
        row0 = pl.multiple_of(g * CHUNK, CHUNK)
        return pltpu.make_async_copy(
            table_hbm.at[idx_v.at[pl.ds(row0, CHUNK)]],
            rows_v.at[b],
            sem_in.at[b],
        )

    def out_copy(g, b):
        row0 = pl.multiple_of(g * CHUNK, CHUNK)
        return pltpu.make_async_copy(
            rows_v.at[b],
            out_hbm.at[pl.ds(base + row0, CHUNK)],
            sem_out.at[b],
        )


    gather_copy(0, 0).start()
    gather_copy(1, 1).start()
    gather_copy(2, 2).start()
    gather_copy(0, 0).wait()
    out_copy(0, 0).start()
    gather_copy(3, 3).start()
    gather_copy(1, 1).wait()
    out_copy(1, 1).start()
    out_copy(0, 0).wait()
    gather_copy(4, 0).start()
    gather_copy(2, 2).wait()
    out_copy(2, 2).start()
    out_copy(1, 1).wait()
    gather_copy(5, 1).start()
    gather_copy(3, 3).wait()
    out_copy(3, 3).start()

    def body(gi, carry):
        g0 = gi * NBUF
        for b in range(NBUF):
            g = g0 + b
            b2 = (b + 2) % NBUF
            out_copy(g - 2, b2).wait()
            gather_copy(g + 2, b2).start()
            gather_copy(g, b).wait()
            out_copy(g, b).start()
        return carry

    lax.fori_loop(1, N_CHUNKS // NBUF - 1, body, 0)

    gl = N_CHUNKS - NBUF
    out_copy(gl - 2, 2).wait()
    gather_copy(gl + 2, 2).start()
    gather_copy(gl, 0).wait()
    out_copy(gl, 0).start()
    out_copy(gl - 1, 3).wait()
    gather_copy(gl + 3, 3).start()
    gather_copy(gl + 1, 1).wait()
    out_copy(gl + 1, 1).start()
    gather_copy(gl + 2, 2).wait()
    out_copy(gl + 2, 2).start()
    gather_copy(gl + 3, 3).wait()
    out_copy(gl + 3, 3).start()
    for b in range(NBUF):
        out_copy(gl + b, b).wait()


def kernel(x, embed_weight, fc_W, fc_b):
    table = _fused_table(embed_weight, fc_W, fc_b)
    idx = x.reshape(-1).astype(jnp.int32)
    out = _sc_gather(table, idx)
    return out.reshape(x.shape[0], x.shape[1], DIM)


# speedup vs baseline: 17.8734x; 4.8148x over previous
import functools

import jax
import jax.numpy as jnp
from jax import lax
from jax.experimental import pallas as pl
from jax.experimental.pallas import tpu as pltpu
from jax.experimental.pallas import tpu_sc as plsc

VOCAB = 100
DIM = 128

NC = 2
NS = 16
NW = NC * NS

B_TOKENS = 4096 * 200
B_PER_W = B_TOKENS // NW
CHUNK = 128
N_CHUNKS = B_PER_W // CHUNK
NBUF = 4


def _table_body(ew_ref, w_ref, b_ref, out_ref):
    ew = ew_ref[...]
    w = w_ref[...]
    out_ref[...] = (
        lax.dot_general(ew, w, (((1,), (1,)), ((), ())),
                        preferred_element_type=jnp.float32)
        + b_ref[...]
    )


def _fused_table(embed_weight, fc_W, fc_b):
    return pl.pallas_call(
        _table_body,
        out_shape=jax.ShapeDtypeStruct((VOCAB, DIM), jnp.float32),
    )(embed_weight, fc_W, fc_b.reshape(1, DIM))


_sc_mesh = plsc.VectorSubcoreMesh(
    core_axis_name="c", subcore_axis_name="s", num_cores=NC, num_subcores=NS
)


@functools.partial(
    pl.kernel,
    out_type=jax.ShapeDtypeStruct((B_TOKENS, DIM), jnp.float32),
    mesh=_sc_mesh,
    scratch_types=[
        pltpu.VMEM((B_PER_W,), jnp.int32),
        pltpu.VMEM((NBUF, CHUNK, DIM), jnp.float32),
        pltpu.VMEM_SHARED((VOCAB, DIM), jnp.float32),
        pltpu.SemaphoreType.DMA((NBUF,)),
        pltpu.SemaphoreType.DMA((NBUF,)),
    ],
)
def _sc_gather(table_hbm, idx_hbm, out_hbm, idx_v, rows_v, table_sp,
               sem_in, sem_out):
    sid = lax.axis_index("s")
    wid = sid * NC + lax.axis_index("c")
    base = wid * B_PER_W

    @pl.when(sid == 0)
    def _():
        pltpu.sync_copy(table_hbm, table_sp)

    pltpu.sync_copy(idx_hbm.at[pl.ds(base, B_PER_W)], idx_v)
    plsc.subcore_barrier()

    def gather_copy(g, b):
        row0 = pl.multiple_of(g * CHUNK, CHUNK)
        return pltpu.make_async_copy(
            table_sp.at[idx_v.at[pl.ds(row0, CHUNK)]],
            rows_v.at[b],
            sem_in.at[b],
        )

    def out_copy(g, b):
        row0 = pl.multiple_of(g * CHUNK, CHUNK)
        return pltpu.make_async_copy(
            rows_v.at[b],
            out_hbm.at[pl.ds(base + row0, CHUNK)],
            sem_out.at[b],
        )


    gather_copy(0, 0).start()
    gather_copy(1, 1).start()
    gather_copy(2, 2).start()
    gather_copy(0, 0).wait()
    out_copy(0, 0).start()
    gather_copy(3, 3).start()
    gather_copy(1, 1).wait()
    out_copy(1, 1).start()
    out_copy(0, 0).wait()
    gather_copy(4, 0).start()
    gather_copy(2, 2).wait()
    out_copy(2, 2).start()
    out_copy(1, 1).wait()
    gather_copy(5, 1).start()
    gather_copy(3, 3).wait()
    out_copy(3, 3).start()

    def body(gi, carry):
        g0 = gi * NBUF
        for b in range(NBUF):
            g = g0 + b
            b2 = (b + 2) % NBUF
            out_copy(g - 2, b2).wait()
            gather_copy(g + 2, b2).start()
            gather_copy(g, b).wait()
            out_copy(g, b).start()
        return carry

    lax.fori_loop(1, N_CHUNKS // NBUF - 1, body, 0)

    gl = N_CHUNKS - NBUF
    out_copy(gl - 2, 2).wait()
    gather_copy(gl + 2, 2).start()
    gather_copy(gl, 0).wait()
    out_copy(gl, 0).start()
    out_copy(gl - 1, 3).wait()
    gather_copy(gl + 3, 3).start()
    gather_copy(gl + 1, 1).wait()
    out_copy(gl + 1, 1).start()
    gather_copy(gl + 2, 2).wait()
    out_copy(gl + 2, 2).start()
    gather_copy(gl + 3, 3).wait()
    out_copy(gl + 3, 3).start()
    for b in range(NBUF):
        out_copy(gl + b, b).wait()


def kernel(x, embed_weight, fc_W, fc_b):
    table = _fused_table(embed_weight, fc_W, fc_b)
    idx = x.reshape(-1).astype(jnp.int32)
    out = _sc_gather(table, idx)
    return out.reshape(x.shape[0], x.shape[1], DIM)
